# baseline (speedup 1.0000x reference)
import functools

import jax
import jax.numpy as jnp
from jax import lax
from jax.experimental import pallas as pl
from jax.experimental.pallas import tpu as pltpu
from jax.experimental.pallas import tpu_sc as plsc

_BATCH = 16384
_EMBED = 16
_HOUR_VOCAB = 24
_PHONE_VOCAB = 100000
_NC = 2
_NS = 16
_NW = _NC * _NS
_B_PER_W = _BATCH // _NW
_G = 16


_NSLAB = 25
_SLAB = 500
_SLAB_PAD = 504
_MAGIC = 33555


@jax.jit
def _context_embedding_sc(hour_idx, phone_idx, hour_table, pt_slab):
    mesh = plsc.VectorSubcoreMesh(core_axis_name="c", subcore_axis_name="s")

    @functools.partial(
        pl.kernel,
        mesh=mesh,
        out_type=jax.ShapeDtypeStruct((_BATCH, 2 * _EMBED), jnp.float32),
        scratch_types=[
            pltpu.VMEM((_HOUR_VOCAB, _EMBED), jnp.float32),
            pltpu.VMEM((_B_PER_W,), jnp.int32),
            pltpu.VMEM((_B_PER_W,), jnp.int32),
            pltpu.VMEM((_B_PER_W,), jnp.int32),
            pltpu.VMEM((_B_PER_W, 128), jnp.float32),
            pltpu.VMEM((_B_PER_W // 2, 2 * _EMBED), jnp.float32),
            pltpu.SemaphoreType.DMA,
        ],
    )
    def k(hi_hbm, pi_hbm, ht_hbm, pt_hbm, out_hbm,
          ht_v, hi_v, pi_v, psup_v, prows_v, cat_v, sem):
        wid = lax.axis_index("s") * _NC + lax.axis_index("c")
        base = wid * _B_PER_W
        pltpu.sync_copy(hi_hbm.at[pl.ds(base, _B_PER_W)], hi_v)
        pltpu.sync_copy(pi_hbm.at[pl.ds(base, _B_PER_W)], pi_v)
        pltpu.sync_copy(ht_hbm, ht_v)

        @pl.loop(0, _B_PER_W // _G)
        def _(g):
            sr = pi_v[pl.ds(g * _G, _G)] >> 3
            slab = (sr * _MAGIC) >> 24
            psup_v.at[pl.ds(g * _G, _G)][...] = sr + (slab << 2)

        pt2 = pt_hbm.reshape(_NSLAB * _SLAB_PAD, 128)
        gp = pltpu.async_copy(pt2.at[psup_v], prows_v, sem)
        gp.wait()

        half = _B_PER_W // 2
        for s in range(2):
            @pl.loop(0, half // _G)
            def _(g):
                hvec = hi_v[pl.ds(s * half + g * _G, _G)]
                pvec = (pi_v[pl.ds(s * half + g * _G, _G)] & 7) * _EMBED
                for j in range(_G):
                    i = g * _G + j
                    cat_v.at[i, pl.ds(0, _EMBED)][...] = (
                        ht_v.at[hvec[j], pl.ds(0, _EMBED)][...])
                    cat_v.at[i, pl.ds(_EMBED, _EMBED)][...] = (
                        prows_v.at[s * half + i, pl.ds(pvec[j], _EMBED)][...])

            pltpu.sync_copy(cat_v, out_hbm.at[pl.ds(base + s * half, half)])

    return k(hour_idx, phone_idx, hour_table, pt_slab)


def _relayout_body(in_ref, out_ref):
    for j in range(8):
        out_ref[0, pl.ds(0, _SLAB), pl.ds(j * _EMBED, _EMBED)] = (
            in_ref[:, j, :])


def _to_slabs(pt):
    pt3 = pt.reshape(_PHONE_VOCAB // 8, 8, _EMBED)
    return pl.pallas_call(
        _relayout_body,
        out_shape=jax.ShapeDtypeStruct((_NSLAB, _SLAB_PAD, 128), jnp.float32),
        grid=(_NSLAB,),
        in_specs=[pl.BlockSpec((_SLAB, 8, _EMBED), lambda i: (i, 0, 0))],
        out_specs=pl.BlockSpec((1, _SLAB_PAD, 128), lambda i: (i, 0, 0)),
    )(pt3)


def kernel(hour_idx, phone_idx, hour_table, phone_table):
    return _context_embedding_sc(
        hour_idx.astype(jnp.int32),
        phone_idx.astype(jnp.int32),
        hour_table,
        _to_slabs(phone_table),
    )


# speedup vs baseline: 1.0667x; 1.0667x over previous
import functools

import jax
import jax.numpy as jnp
from jax import lax
from jax.experimental import pallas as pl
from jax.experimental.pallas import tpu as pltpu
from jax.experimental.pallas import tpu_sc as plsc

_BATCH = 16384
_EMBED = 16
_HOUR_VOCAB = 24
_PHONE_VOCAB = 100000
_NC = 2
_NS = 16
_NW = _NC * _NS
_B_PER_W = _BATCH // _NW
_G = 16


@jax.jit
def _context_embedding_sc(hour_idx, phone_idx, hour_table, pt_wide):
    mesh = plsc.VectorSubcoreMesh(core_axis_name="c", subcore_axis_name="s")

    @functools.partial(
        pl.kernel,
        mesh=mesh,
        out_type=jax.ShapeDtypeStruct((_BATCH, 2 * _EMBED), jnp.float32),
        scratch_types=[
            pltpu.VMEM((_HOUR_VOCAB, _EMBED), jnp.float32),
            pltpu.VMEM((_B_PER_W,), jnp.int32),
            pltpu.VMEM((_B_PER_W,), jnp.int32),
            pltpu.VMEM((_B_PER_W,), jnp.int32),
            pltpu.VMEM((_B_PER_W, 128), jnp.float32),
            pltpu.VMEM((_B_PER_W // 2, 2 * _EMBED), jnp.float32),
            pltpu.SemaphoreType.DMA,
            pltpu.SemaphoreType.DMA,
        ],
    )
    def k(hi_hbm, pi_hbm, ht_hbm, pt_hbm, out_hbm,
          ht_v, hi_v, pi_v, psup_v, prows_v, cat_v, sem, sem2):
        wid = lax.axis_index("s") * _NC + lax.axis_index("c")
        base = wid * _B_PER_W
        pltpu.sync_copy(hi_hbm.at[pl.ds(base, _B_PER_W)], hi_v)
        pltpu.sync_copy(pi_hbm.at[pl.ds(base, _B_PER_W)], pi_v)
        pltpu.sync_copy(ht_hbm, ht_v)

        @pl.loop(0, _B_PER_W // _G)
        def _(g):
            psup_v.at[pl.ds(g * _G, _G)][...] = (
                pi_v[pl.ds(g * _G, _G)] >> 3)

        half = _B_PER_W // 2
        g0 = pltpu.async_copy(
            pt_hbm.at[psup_v.at[pl.ds(0, half)]],
            prows_v.at[pl.ds(0, half)], sem)
        g1 = pltpu.async_copy(
            pt_hbm.at[psup_v.at[pl.ds(half, half)]],
            prows_v.at[pl.ds(half, half)], sem2)

        waits = (g0, g1)
        for s in range(2):
            waits[s].wait()

            @pl.loop(0, half // _G)
            def _(g):
                hvec = hi_v[pl.ds(s * half + g * _G, _G)]
                pvec = (pi_v[pl.ds(s * half + g * _G, _G)] & 7) * _EMBED
                for j in range(_G):
                    i = g * _G + j
                    cat_v.at[i, pl.ds(0, _EMBED)][...] = (
                        ht_v.at[hvec[j], pl.ds(0, _EMBED)][...])
                    cat_v.at[i, pl.ds(_EMBED, _EMBED)][...] = (
                        prows_v.at[s * half + i, pl.ds(pvec[j], _EMBED)][...])

            pltpu.sync_copy(cat_v, out_hbm.at[pl.ds(base + s * half, half)])

    return k(hour_idx, phone_idx, hour_table, pt_wide)


def kernel(hour_idx, phone_idx, hour_table, phone_table):
    return _context_embedding_sc(
        hour_idx.astype(jnp.int32),
        phone_idx.astype(jnp.int32),
        hour_table,
        phone_table.reshape(_PHONE_VOCAB // 8, 128),
    )


# speedup vs baseline: 1.0734x; 1.0063x over previous
import functools

import jax
import jax.numpy as jnp
from jax import lax
from jax.experimental import pallas as pl
from jax.experimental.pallas import tpu as pltpu
from jax.experimental.pallas import tpu_sc as plsc

_BATCH = 16384
_EMBED = 16
_HOUR_VOCAB = 24
_PHONE_VOCAB = 100000
_NC = 2
_NS = 16
_NW = _NC * _NS
_B_PER_W = _BATCH // _NW
_G = 16


@jax.jit
def _context_embedding_sc(hour_idx, phone_idx, hour_table, pt_wide):
    mesh = plsc.VectorSubcoreMesh(core_axis_name="c", subcore_axis_name="s")

    @functools.partial(
        pl.kernel,
        mesh=mesh,
        out_type=jax.ShapeDtypeStruct((_BATCH, 2 * _EMBED), jnp.float32),
        scratch_types=[
            pltpu.VMEM((_HOUR_VOCAB, _EMBED), jnp.float32),
            pltpu.VMEM((_B_PER_W,), jnp.int32),
            pltpu.VMEM((_B_PER_W,), jnp.int32),
            pltpu.VMEM((_B_PER_W,), jnp.int32),
            pltpu.VMEM((_B_PER_W, 128), jnp.float32),
            pltpu.VMEM((_B_PER_W // 2, 2 * _EMBED), jnp.float32),
            pltpu.SemaphoreType.DMA,
        ],
    )
    def k(hi_hbm, pi_hbm, ht_hbm, pt_hbm, out_hbm,
          ht_v, hi_v, pi_v, psup_v, prows_v, cat_v, sem):
        wid = lax.axis_index("s") * _NC + lax.axis_index("c")
        base = wid * _B_PER_W
        pltpu.sync_copy(hi_hbm.at[pl.ds(base, _B_PER_W)], hi_v)
        pltpu.sync_copy(pi_hbm.at[pl.ds(base, _B_PER_W)], pi_v)
        pltpu.sync_copy(ht_hbm, ht_v)

        @pl.loop(0, _B_PER_W // _G)
        def _(g):
            psup_v.at[pl.ds(g * _G, _G)][...] = (
                pi_v[pl.ds(g * _G, _G)] >> 3)

        gp = pltpu.async_copy(pt_hbm.at[psup_v], prows_v, sem)
        gp.wait()

        half = _B_PER_W // 2
        for s in range(2):
            @pl.loop(0, half // _G)
            def _(g):
                hvec = hi_v[pl.ds(s * half + g * _G, _G)]
                pvec = (pi_v[pl.ds(s * half + g * _G, _G)] & 7) * _EMBED
                for j in range(_G):
                    i = g * _G + j
                    cat_v.at[i, pl.ds(0, _EMBED)][...] = (
                        ht_v.at[hvec[j], pl.ds(0, _EMBED)][...])
                    cat_v.at[i, pl.ds(_EMBED, _EMBED)][...] = (
                        prows_v.at[s * half + i, pl.ds(pvec[j], _EMBED)][...])

            pltpu.sync_copy(cat_v, out_hbm.at[pl.ds(base + s * half, half)])

    return k(hour_idx, phone_idx, hour_table, pt_wide)


def kernel(hour_idx, phone_idx, hour_table, phone_table):
    return _context_embedding_sc(
        hour_idx.astype(jnp.int32),
        phone_idx.astype(jnp.int32),
        hour_table,
        phone_table.reshape(_PHONE_VOCAB // 8, 128),
    )
